# Initial kernel scaffold; baseline (speedup 1.0000x reference)
import functools

import jax
import jax.numpy as jnp
import numpy as np
from jax import lax
from jax.experimental import pallas as pl
from jax.experimental.pallas import tpu as pltpu
from jax.experimental.pallas import tpu_sc as plsc

_B, _N = 64, 576
_LANES = 16


def _build_index_constants():
    noise = np.asarray(
        jax.random.uniform(jax.random.key(42), (_B, _N), dtype=jnp.float32)
    )
    shuffle = np.argsort(noise, axis=-1, kind="stable").astype(np.int32)
    revert = np.argsort(shuffle, axis=-1, kind="stable").astype(np.int32)
    return shuffle, revert


_SHUFFLE_NP, _REVERT_NP = _build_index_constants()


@functools.partial(jax.jit, static_argnums=(5, 6, 7, 8))
def _sc_gather_add(data2, pe, gtok, didx, pidx, B, N, D, num_remain):
    info = plsc.get_sparse_core_info()
    NC, NS = info.num_cores, info.num_subcores
    NW = NC * NS
    BPW = B // NW
    CH = 48
    NCHUNK = num_remain // CH
    NV = D // _LANES

    mesh = plsc.VectorSubcoreMesh(core_axis_name="c", subcore_axis_name="s")

    @functools.partial(
        pl.kernel,
        out_type=jax.ShapeDtypeStruct((B, num_remain + 1, D), jnp.float32),
        mesh=mesh,
        scratch_types=[
            pltpu.VMEM((num_remain,), jnp.int32),
            pltpu.VMEM((num_remain,), jnp.int32),
            pltpu.VMEM((CH, D), jnp.float32),
            pltpu.VMEM((CH, D), jnp.float32),
            pltpu.VMEM((D,), jnp.float32),
            pltpu.VMEM((D,), jnp.float32),
            pltpu.SemaphoreType.DMA,
            pltpu.SemaphoreType.DMA,
        ],
    )
    def k(data2_h, pe_h, gt_h, didx_h, pidx_h, out_h,
          didx_v, pidx_v, pe_buf, dat_buf, gt_v, pe0_v, sem_a, sem_b):
        wid = lax.axis_index("s") * NC + lax.axis_index("c")

        pltpu.sync_copy(gt_h.at[0], gt_v)
        pltpu.sync_copy(pe_h.at[0], pe0_v)

        def _gt_add(i, carry):
            plsc.addupdate(gt_v.at[pl.ds(i * _LANES, _LANES)],
                           pe0_v[pl.ds(i * _LANES, _LANES)])
            return carry

        lax.fori_loop(0, NV, _gt_add, 0)

        for t in range(BPW):
            b = wid * BPW + t
            pltpu.sync_copy(gt_v, out_h.at[b, 0])
            pltpu.sync_copy(didx_h.at[pl.ds(b * num_remain, num_remain)], didx_v)
            pltpu.sync_copy(pidx_h.at[pl.ds(b * num_remain, num_remain)], pidx_v)
            for ci in range(NCHUNK):
                cp_pe = pltpu.async_copy(
                    pe_h.at[pidx_v.at[pl.ds(ci * CH, CH)]], pe_buf, sem_a)
                cp_dat = pltpu.async_copy(
                    data2_h.at[didx_v.at[pl.ds(ci * CH, CH)]], dat_buf, sem_b)
                cp_pe.wait()
                cp_dat.wait()

                def _row_add(r, carry):
                    def _vreg_add(kk, c2):
                        plsc.addupdate(
                            pe_buf.at[r, pl.ds(kk * _LANES, _LANES)],
                            dat_buf[r, pl.ds(kk * _LANES, _LANES)])
                        return c2
                    lax.fori_loop(0, NV, _vreg_add, 0)
                    return carry

                lax.fori_loop(0, CH, _row_add, 0)
                pltpu.sync_copy(pe_buf, out_h.at[b, pl.ds(1 + ci * CH, CH)])

    return k(data2, pe, gtok, didx, pidx)


def kernel(data, remain_rto, pos_enc, global_token):
    B, N, D = data.shape
    num_remain = int(N * 0.25)
    assert (B, N) == (_B, _N)

    shuffle_np, revert_np = _SHUFFLE_NP, _REVERT_NP
    remain_np = shuffle_np[:, :num_remain]
    didx_np = (np.arange(B, dtype=np.int32)[:, None] * N + remain_np).reshape(-1)
    pidx_np = (remain_np + 1).reshape(-1)

    data2 = data.reshape(B * N, D)
    out = _sc_gather_add(
        data2, pos_enc, global_token,
        jnp.asarray(didx_np), jnp.asarray(pidx_np),
        B, N, D, num_remain)

    return (out,
            jnp.asarray(remain_np),
            jnp.asarray(shuffle_np[:, num_remain:]),
            jnp.asarray(revert_np))


# speedup vs baseline: 1.0269x; 1.0269x over previous
import functools

import jax
import jax.numpy as jnp
import numpy as np
from jax import lax
from jax.experimental import pallas as pl
from jax.experimental.pallas import tpu as pltpu
from jax.experimental.pallas import tpu_sc as plsc

_B, _N = 64, 576
_LANES = 16
_IDX_STRIDE = 152

_G_SIZES = (48, 48, 56)
_D_FILL = (48, 48, 49)
_P_FILL = (48, 48, 49)
_D_REAL = (47, 48, 49)
_C_STARTS = (0, 48, 96)
_O_SIZE = 48


def _build_index_constants():
    noise = np.asarray(
        jax.random.uniform(jax.random.key(42), (_B, _N), dtype=jnp.float32)
    )
    shuffle = np.argsort(noise, axis=-1, kind="stable").astype(np.int32)
    revert = np.argsort(shuffle, axis=-1, kind="stable").astype(np.int32)
    return shuffle, revert


_SHUFFLE_NP, _REVERT_NP = _build_index_constants()


def _build_gather_lists(remain_np, num_remain):
    B = remain_np.shape[0]
    didx = np.zeros((B, _IDX_STRIDE), dtype=np.int32)
    pidx = np.zeros((B, _IDX_STRIDE), dtype=np.int32)
    rows = np.arange(B, dtype=np.int32)[:, None] * _N + remain_np
    pe_rows = remain_np + 1
    g = 0
    for c in range(3):
        s = _C_STARTS[c]
        didx[:, s:s + _D_FILL[c]] = rows[:, g:g + _D_FILL[c]]
        if c == 0:
            pidx[:, s] = 0
            pidx[:, s + 1:s + _P_FILL[c]] = pe_rows[:, g:g + _P_FILL[c] - 1]
        else:
            pidx[:, s:s + _P_FILL[c]] = pe_rows[:, g:g + _P_FILL[c]]
        g += _D_REAL[c]
    assert g == num_remain
    return didx.reshape(-1), pidx.reshape(-1)


@functools.partial(jax.jit, static_argnums=(5, 6, 7, 8))
def _sc_gather_add(data2, pe, gtok, didx, pidx, B, N, D, num_remain):
    info = plsc.get_sparse_core_info()
    NC, NS = info.num_cores, info.num_subcores
    NW = NC * NS
    BPW = B // NW
    NV = D // _LANES
    BUF_ROWS = max(_G_SIZES)

    mesh = plsc.VectorSubcoreMesh(core_axis_name="c", subcore_axis_name="s")

    @functools.partial(
        pl.kernel,
        out_type=jax.ShapeDtypeStruct((B, num_remain + 1, D), jnp.float32),
        mesh=mesh,
        scratch_types=[
            pltpu.VMEM((_IDX_STRIDE,), jnp.int32),
            pltpu.VMEM((_IDX_STRIDE,), jnp.int32),
            pltpu.VMEM((BUF_ROWS, D), jnp.float32),
            pltpu.VMEM((BUF_ROWS, D), jnp.float32),
            pltpu.VMEM((D,), jnp.float32),
            pltpu.VMEM((1, D), jnp.float32),
            pltpu.SemaphoreType.DMA,
            pltpu.SemaphoreType.DMA,
        ],
    )
    def k(data2_h, pe_h, gt_h, didx_h, pidx_h, out_h,
          didx_v, pidx_v, pe_buf, dat_buf, gt_v, lrow_buf, sem_a, sem_b):
        wid = lax.axis_index("s") * NC + lax.axis_index("c")

        pltpu.sync_copy(gt_h.at[0], gt_v)

        for t in range(BPW):
            b = wid * BPW + t
            pltpu.sync_copy(didx_h.at[pl.ds(b * _IDX_STRIDE, _IDX_STRIDE)],
                            didx_v)
            pltpu.sync_copy(pidx_h.at[pl.ds(b * _IDX_STRIDE, _IDX_STRIDE)],
                            pidx_v)
            for c in range(3):
                s = _C_STARTS[c]
                gsz, dsz = _G_SIZES[c], _D_REAL[c]
                shift = 1 if c == 0 else 0
                pe_dst = pe_buf if gsz == BUF_ROWS else pe_buf.at[pl.ds(0, gsz)]
                dat_dst = (dat_buf if gsz == BUF_ROWS
                           else dat_buf.at[pl.ds(0, gsz)])
                cp_pe = pltpu.async_copy(
                    pe_h.at[pidx_v.at[pl.ds(s, gsz)]], pe_dst, sem_a)
                cp_dat = pltpu.async_copy(
                    data2_h.at[didx_v.at[pl.ds(s, gsz)]], dat_dst, sem_b)
                cp_pe.wait()
                cp_dat.wait()

                if c == 0:
                    def _gt_add(i, carry):
                        plsc.addupdate(
                            pe_buf.at[0, pl.ds(i * _LANES, _LANES)],
                            gt_v[pl.ds(i * _LANES, _LANES)])
                        return carry
                    lax.fori_loop(0, NV, _gt_add, 0)

                def _row_add(r, carry):
                    def _vreg_add(kk, c2):
                        plsc.addupdate(
                            pe_buf.at[r + shift, pl.ds(kk * _LANES, _LANES)],
                            dat_buf[r, pl.ds(kk * _LANES, _LANES)])
                        return c2
                    lax.fori_loop(0, NV, _vreg_add, 0)
                    return carry

                lax.fori_loop(0, dsz, _row_add, 0)

                pltpu.sync_copy(pe_buf.at[pl.ds(0, _O_SIZE)],
                                out_h.at[b, pl.ds(s, _O_SIZE)])
                if c == 2:
                    def _lrow_copy(i, carry):
                        lrow_buf[0, pl.ds(i * _LANES, _LANES)] = (
                            pe_buf[48, pl.ds(i * _LANES, _LANES)])
                        return carry
                    lax.fori_loop(0, NV, _lrow_copy, 0)
                    pltpu.sync_copy(lrow_buf, out_h.at[b, pl.ds(144, 1)])

    return k(data2, pe, gtok, didx, pidx)


def kernel(data, remain_rto, pos_enc, global_token):
    B, N, D = data.shape
    num_remain = int(N * 0.25)
    assert (B, N) == (_B, _N)

    shuffle_np, revert_np = _SHUFFLE_NP, _REVERT_NP
    remain_np = shuffle_np[:, :num_remain]
    didx_np, pidx_np = _build_gather_lists(remain_np, num_remain)

    data2 = data.reshape(B * N, D)
    out = _sc_gather_add(
        data2, pos_enc, global_token,
        jnp.asarray(didx_np), jnp.asarray(pidx_np),
        B, N, D, num_remain)

    return (out,
            jnp.asarray(remain_np),
            jnp.asarray(shuffle_np[:, num_remain:]),
            jnp.asarray(revert_np))


# speedup vs baseline: 1.1967x; 1.1654x over previous
import functools

import jax
import jax.numpy as jnp
import numpy as np
from jax import lax
from jax.experimental import pallas as pl
from jax.experimental.pallas import tpu as pltpu
from jax.experimental.pallas import tpu_sc as plsc

_B, _N = 64, 576
_LANES = 16
_IDX_STRIDE = 152

_G_SIZES = (48, 48, 56)
_D_FILL = (48, 48, 49)
_P_FILL = (48, 48, 49)
_D_REAL = (47, 48, 49)
_C_STARTS = (0, 48, 96)
_O_SIZE = 48


def _build_index_constants():
    noise = np.asarray(
        jax.random.uniform(jax.random.key(42), (_B, _N), dtype=jnp.float32)
    )
    shuffle = np.argsort(noise, axis=-1, kind="stable").astype(np.int32)
    revert = np.argsort(shuffle, axis=-1, kind="stable").astype(np.int32)
    return shuffle, revert


_SHUFFLE_NP, _REVERT_NP = _build_index_constants()


def _build_gather_lists(remain_np, num_remain):
    B = remain_np.shape[0]
    didx = np.zeros((B, _IDX_STRIDE), dtype=np.int32)
    pidx = np.zeros((B, _IDX_STRIDE), dtype=np.int32)
    rows = np.arange(B, dtype=np.int32)[:, None] * _N + remain_np
    pe_rows = remain_np + 1
    g = 0
    for c in range(3):
        s = _C_STARTS[c]
        didx[:, s:s + _D_FILL[c]] = rows[:, g:g + _D_FILL[c]]
        if c == 0:
            pidx[:, s] = 0
            pidx[:, s + 1:s + _P_FILL[c]] = pe_rows[:, g:g + _P_FILL[c] - 1]
        else:
            pidx[:, s:s + _P_FILL[c]] = pe_rows[:, g:g + _P_FILL[c]]
        g += _D_REAL[c]
    assert g == num_remain
    return didx.reshape(-1), pidx.reshape(-1)


@functools.partial(jax.jit, static_argnums=(5, 6, 7, 8))
def _sc_gather_add(data2, pe, gtok, didx, pidx, B, N, D, num_remain):
    info = plsc.get_sparse_core_info()
    NC, NS = info.num_cores, info.num_subcores
    NW = NC * NS
    BPW = B // NW
    NV = D // _LANES
    BUF_ROWS = max(_G_SIZES)

    mesh = plsc.VectorSubcoreMesh(core_axis_name="c", subcore_axis_name="s")

    @functools.partial(
        pl.kernel,
        out_type=jax.ShapeDtypeStruct((B, num_remain + 1, D), jnp.float32),
        mesh=mesh,
        scratch_types=[
            pltpu.VMEM((_IDX_STRIDE,), jnp.int32),
            pltpu.VMEM((_IDX_STRIDE,), jnp.int32),
            pltpu.VMEM((BUF_ROWS, D), jnp.float32),
            pltpu.VMEM((BUF_ROWS, D), jnp.float32),
            pltpu.VMEM((D,), jnp.float32),
            pltpu.VMEM((1, D), jnp.float32),
            pltpu.SemaphoreType.DMA,
            pltpu.SemaphoreType.DMA,
        ],
    )
    def k(data2_h, pe_h, gt_h, didx_h, pidx_h, out_h,
          didx_v, pidx_v, pe_buf, dat_buf, gt_v, lrow_buf, sem_a, sem_b):
        wid = lax.axis_index("s") * NC + lax.axis_index("c")

        pltpu.sync_copy(gt_h.at[0], gt_v)

        for t in range(BPW):
            b = wid * BPW + t
            pltpu.sync_copy(didx_h.at[pl.ds(b * _IDX_STRIDE, _IDX_STRIDE)],
                            didx_v)
            pltpu.sync_copy(pidx_h.at[pl.ds(b * _IDX_STRIDE, _IDX_STRIDE)],
                            pidx_v)
            for c in range(3):
                s = _C_STARTS[c]
                gsz, dsz = _G_SIZES[c], _D_REAL[c]
                shift = 1 if c == 0 else 0
                pe_dst = pe_buf if gsz == BUF_ROWS else pe_buf.at[pl.ds(0, gsz)]
                dat_dst = (dat_buf if gsz == BUF_ROWS
                           else dat_buf.at[pl.ds(0, gsz)])
                cp_pe = pltpu.async_copy(
                    pe_h.at[pidx_v.at[pl.ds(s, gsz)]], pe_dst, sem_a)
                cp_dat = pltpu.async_copy(
                    data2_h.at[didx_v.at[pl.ds(s, gsz)]], dat_dst, sem_b)
                cp_pe.wait()
                cp_dat.wait()

                if c == 0:
                    for i in range(NV):
                        plsc.addupdate(
                            pe_buf.at[0, pl.ds(i * _LANES, _LANES)],
                            gt_v[pl.ds(i * _LANES, _LANES)])

                def _row_add(r, carry):
                    for kk in range(NV):
                        plsc.addupdate(
                            pe_buf.at[r + shift, pl.ds(kk * _LANES, _LANES)],
                            dat_buf[r, pl.ds(kk * _LANES, _LANES)])
                    return carry

                lax.fori_loop(0, dsz, _row_add, 0)

                pltpu.sync_copy(pe_buf.at[pl.ds(0, _O_SIZE)],
                                out_h.at[b, pl.ds(s, _O_SIZE)])
                if c == 2:
                    for i in range(NV):
                        lrow_buf[0, pl.ds(i * _LANES, _LANES)] = (
                            pe_buf[48, pl.ds(i * _LANES, _LANES)])
                    pltpu.sync_copy(lrow_buf, out_h.at[b, pl.ds(144, 1)])

    return k(data2, pe, gtok, didx, pidx)


def kernel(data, remain_rto, pos_enc, global_token):
    B, N, D = data.shape
    num_remain = int(N * 0.25)
    assert (B, N) == (_B, _N)

    shuffle_np, revert_np = _SHUFFLE_NP, _REVERT_NP
    remain_np = shuffle_np[:, :num_remain]
    didx_np, pidx_np = _build_gather_lists(remain_np, num_remain)

    data2 = data.reshape(B * N, D)
    out = _sc_gather_add(
        data2, pos_enc, global_token,
        jnp.asarray(didx_np), jnp.asarray(pidx_np),
        B, N, D, num_remain)

    return (out,
            jnp.asarray(remain_np),
            jnp.asarray(shuffle_np[:, num_remain:]),
            jnp.asarray(revert_np))


# speedup vs baseline: 1.3534x; 1.1309x over previous
import functools

import jax
import jax.numpy as jnp
import numpy as np
from jax import lax
from jax.experimental import pallas as pl
from jax.experimental.pallas import tpu as pltpu
from jax.experimental.pallas import tpu_sc as plsc

_B, _N = 64, 576
_LANES = 16

_NCHUNK = 6
_OUT_CH = 24
_CH_SPACE = 32
_G_SIZES = (24, 24, 24, 24, 24, 32)
_FILLS = (24, 24, 24, 24, 24, 25)
_D_REAL = (23, 24, 24, 24, 24, 25)
_IDX_STRIDE = _NCHUNK * _CH_SPACE


def _build_index_constants():
    noise = np.asarray(
        jax.random.uniform(jax.random.key(42), (_B, _N), dtype=jnp.float32)
    )
    shuffle = np.argsort(noise, axis=-1, kind="stable").astype(np.int32)
    revert = np.argsort(shuffle, axis=-1, kind="stable").astype(np.int32)
    return shuffle, revert


_SHUFFLE_NP, _REVERT_NP = _build_index_constants()


def _build_gather_lists(remain_np, num_remain):
    B = remain_np.shape[0]
    didx = np.zeros((B, _IDX_STRIDE), dtype=np.int32)
    pidx = np.zeros((B, _IDX_STRIDE), dtype=np.int32)
    rows = np.arange(B, dtype=np.int32)[:, None] * _N + remain_np
    pe_rows = remain_np + 1
    g = 0
    for c in range(_NCHUNK):
        s = _CH_SPACE * c
        didx[:, s:s + _FILLS[c]] = rows[:, g:g + _FILLS[c]]
        if c == 0:
            pidx[:, s] = 0
            pidx[:, s + 1:s + _FILLS[c]] = pe_rows[:, g:g + _FILLS[c] - 1]
        else:
            pidx[:, s:s + _FILLS[c]] = pe_rows[:, g:g + _FILLS[c]]
        g += _D_REAL[c]
    assert g == num_remain
    return didx.reshape(-1), pidx.reshape(-1)


@functools.partial(jax.jit, static_argnums=(5, 6, 7, 8))
def _sc_gather_add(data2, pe, gtok, didx, pidx, B, N, D, num_remain):
    info = plsc.get_sparse_core_info()
    NC, NS = info.num_cores, info.num_subcores
    NW = NC * NS
    BPW = B // NW
    NV = D // _LANES
    BUF_ROWS = max(_G_SIZES)
    T = BPW * _NCHUNK

    mesh = plsc.VectorSubcoreMesh(core_axis_name="c", subcore_axis_name="s")

    @functools.partial(
        pl.kernel,
        out_type=jax.ShapeDtypeStruct((B, num_remain + 1, D), jnp.float32),
        mesh=mesh,
        scratch_types=[
            pltpu.VMEM((BPW * _IDX_STRIDE,), jnp.int32),
            pltpu.VMEM((BPW * _IDX_STRIDE,), jnp.int32),
            pltpu.VMEM((BUF_ROWS, D), jnp.float32),
            pltpu.VMEM((BUF_ROWS, D), jnp.float32),
            pltpu.VMEM((BUF_ROWS, D), jnp.float32),
            pltpu.VMEM((BUF_ROWS, D), jnp.float32),
            pltpu.VMEM((BUF_ROWS, D), jnp.float32),
            pltpu.VMEM((D,), jnp.float32),
            pltpu.VMEM((1, D), jnp.float32),
            pltpu.SemaphoreType.DMA,
            pltpu.SemaphoreType.DMA,
            pltpu.SemaphoreType.DMA,
            pltpu.SemaphoreType.DMA,
            pltpu.SemaphoreType.DMA,
            pltpu.SemaphoreType.DMA,
            pltpu.SemaphoreType.DMA,
            pltpu.SemaphoreType.DMA,
        ],
    )
    def k(data2_h, pe_h, gt_h, didx_h, pidx_h, out_h,
          didx_v, pidx_v, pe0, pe1, pe2, da0, da1, gt_v, lrow_buf,
          gp0, gp1, gp2, gd0, gd1, w0, w1, w2):
        pe_slots = (pe0, pe1, pe2)
        dat_slots = (da0, da1)
        sem_gp = (gp0, gp1, gp2)
        sem_gd = (gd0, gd1)
        sem_w = (w0, w1, w2)

        wid = lax.axis_index("s") * NC + lax.axis_index("c")

        pltpu.sync_copy(gt_h.at[0], gt_v)
        pltpu.sync_copy(
            didx_h.at[pl.ds(wid * BPW * _IDX_STRIDE, BPW * _IDX_STRIDE)],
            didx_v)
        pltpu.sync_copy(
            pidx_h.at[pl.ds(wid * BPW * _IDX_STRIDE, BPW * _IDX_STRIDE)],
            pidx_v)

        pe_cp = [None, None, None]
        dat_cp = [None, None]
        w_cp = [None, None, None]

        def gstart(t):
            bt, c = divmod(t, _NCHUNK)
            off = bt * _IDX_STRIDE + _CH_SPACE * c
            gsz = _G_SIZES[c]
            ps, dsl = t % 3, t % 2
            pe_dst = (pe_slots[ps] if gsz == BUF_ROWS
                      else pe_slots[ps].at[pl.ds(0, gsz)])
            dat_dst = (dat_slots[dsl] if gsz == BUF_ROWS
                       else dat_slots[dsl].at[pl.ds(0, gsz)])
            pe_cp[ps] = pltpu.async_copy(
                pe_h.at[pidx_v.at[pl.ds(off, gsz)]], pe_dst, sem_gp[ps])
            dat_cp[dsl] = pltpu.async_copy(
                data2_h.at[didx_v.at[pl.ds(off, gsz)]], dat_dst, sem_gd[dsl])

        gstart(0)
        gstart(1)
        for t in range(T):
            bt, c = divmod(t, _NCHUNK)
            b = wid * BPW + bt
            ps, dsl = t % 3, t % 2
            pe_b, dat_b = pe_slots[ps], dat_slots[dsl]
            pe_cp[ps].wait()
            dat_cp[dsl].wait()

            if c == 0:
                for i in range(NV):
                    plsc.addupdate(pe_b.at[0, pl.ds(i * _LANES, _LANES)],
                                   gt_v[pl.ds(i * _LANES, _LANES)])

            shift = 1 if c == 0 else 0

            def _row_add(r, carry, pe_b=pe_b, dat_b=dat_b, shift=shift):
                for kk in range(NV):
                    plsc.addupdate(
                        pe_b.at[r + shift, pl.ds(kk * _LANES, _LANES)],
                        dat_b[r, pl.ds(kk * _LANES, _LANES)])
                return carry

            lax.fori_loop(0, _D_REAL[c], _row_add, 0)

            if c == _NCHUNK - 1:
                for i in range(NV):
                    lrow_buf[0, pl.ds(i * _LANES, _LANES)] = (
                        pe_b[_OUT_CH, pl.ds(i * _LANES, _LANES)])
                pltpu.sync_copy(lrow_buf,
                                out_h.at[b, pl.ds(_NCHUNK * _OUT_CH, 1)])

            w_cp[ps] = pltpu.async_copy(
                pe_b.at[pl.ds(0, _OUT_CH)],
                out_h.at[b, pl.ds(c * _OUT_CH, _OUT_CH)], sem_w[ps])

            if t + 2 < T:
                pw = (t + 2) % 3
                if w_cp[pw] is not None:
                    w_cp[pw].wait()
                    w_cp[pw] = None
                gstart(t + 2)

        for s in range(3):
            if w_cp[s] is not None:
                w_cp[s].wait()

    return k(data2, pe, gtok, didx, pidx)


def kernel(data, remain_rto, pos_enc, global_token):
    B, N, D = data.shape
    num_remain = int(N * 0.25)
    assert (B, N) == (_B, _N)

    shuffle_np, revert_np = _SHUFFLE_NP, _REVERT_NP
    remain_np = shuffle_np[:, :num_remain]
    didx_np, pidx_np = _build_gather_lists(remain_np, num_remain)

    data2 = data.reshape(B * N, D)
    out = _sc_gather_add(
        data2, pos_enc, global_token,
        jnp.asarray(didx_np), jnp.asarray(pidx_np),
        B, N, D, num_remain)

    return (out,
            jnp.asarray(remain_np),
            jnp.asarray(shuffle_np[:, num_remain:]),
            jnp.asarray(revert_np))
